# baseline (speedup 1.0000x reference)
import jax
import jax.numpy as jnp
from jax import lax
from jax.experimental import pallas as pl
from jax.experimental.pallas import tpu as pltpu
from jax.experimental.pallas import tpu_sc as plsc

N_VERTS = 100000
N_PAD = 102400
N_JOINTS = 64
K_INFL = 4
V_PER_TILE = N_PAD // 32
V_GROUPS = V_PER_TILE // 16

N_LAP = 1600000
ROW = 128
N_TILES = 32
ROWS_PER_TILE = 416
E_PAD = N_TILES * ROWS_PER_TILE * ROW
CHUNK_ROWS = 16
N_CHUNKS = ROWS_PER_TILE // CHUNK_ROWS
ACC_SLICE = N_PAD // 16


def _mesh():
    return plsc.VectorSubcoreMesh(core_axis_name="c", subcore_axis_name="s",
                                  num_cores=2, num_subcores=16)


def _skin_body(jt_hbm, v_hbm, w_hbm, i_hbm, out_hbm,
               jtbl, vs0, vs1, vs2, ws0, ws1, ws2, ws3,
               is0, is1, is2, is3, ps0, ps1, ps2):
    ci = lax.axis_index("c")
    si = lax.axis_index("s")
    wid = ci * 16 + si
    v0 = wid * V_PER_TILE

    pltpu.sync_copy(jt_hbm, jtbl)
    for c, ref in enumerate((vs0, vs1, vs2)):
        pltpu.sync_copy(v_hbm.at[c, 0, pl.ds(v0, V_PER_TILE)], ref)
    for k, ref in enumerate((ws0, ws1, ws2, ws3)):
        pltpu.sync_copy(w_hbm.at[k, 0, pl.ds(v0, V_PER_TILE)], ref)
    for k, ref in enumerate((is0, is1, is2, is3)):
        pltpu.sync_copy(i_hbm.at[k, 0, pl.ds(v0, V_PER_TILE)], ref)
    wrefs = (ws0, ws1, ws2, ws3)
    irefs = (is0, is1, is2, is3)

    def group(g, carry):
        sl = pl.ds(g * 16, 16)
        x = vs0[sl]
        y = vs1[sl]
        z = vs2[sl]
        w = [wrefs[k][sl] for k in range(K_INFL)]
        s = ((w[0] + w[1]) + (w[2] + w[3])) + 1e-8
        rs = 1.0 / s
        acc0 = jnp.zeros((16,), jnp.float32)
        acc1 = jnp.zeros((16,), jnp.float32)
        acc2 = jnp.zeros((16,), jnp.float32)
        for k in range(K_INFL):
            jk = irefs[k][sl]
            jb = jk * 16
            wk = w[k] * rs
            e = [plsc.load_gather(jtbl, [jb + t]) for t in range(12)]
            acc0 = acc0 + wk * (e[0] * x + e[1] * y + e[2] * z + e[3])
            acc1 = acc1 + wk * (e[4] * x + e[5] * y + e[6] * z + e[7])
            acc2 = acc2 + wk * (e[8] * x + e[9] * y + e[10] * z + e[11])
        ps0[sl] = acc0
        ps1[sl] = acc1
        ps2[sl] = acc2
        return carry

    lax.fori_loop(0, V_GROUPS, group, 0)
    for c, ps in enumerate((ps0, ps1, ps2)):
        pltpu.sync_copy(ps, out_hbm.at[c, 0, pl.ds(v0, V_PER_TILE)])


def _skin_call(jt_flat, v_flat, w_flat, i_flat):
    f = pl.kernel(
        _skin_body,
        out_type=jax.ShapeDtypeStruct((3, 1, N_PAD), jnp.float32),
        mesh=_mesh(),
        compiler_params=pltpu.CompilerParams(needs_layout_passes=False),
        scratch_types=(
            [pltpu.VMEM((N_JOINTS * 16,), jnp.float32)]
            + [pltpu.VMEM((V_PER_TILE,), jnp.float32) for _ in range(3)]
            + [pltpu.VMEM((V_PER_TILE,), jnp.float32) for _ in range(4)]
            + [pltpu.VMEM((V_PER_TILE,), jnp.int32) for _ in range(4)]
            + [pltpu.VMEM((V_PER_TILE,), jnp.float32) for _ in range(3)]
        ),
    )
    return f(jt_flat, v_flat, w_flat, i_flat)


def _lap_body(posed_hbm, edges_hbm, out_hbm,
              pc, eb0, eb1, valb, zb, acc, sem_in0, sem_in1, sem_sc):
    ci = lax.axis_index("c")
    si = lax.axis_index("s")
    wid = ci * 16 + si

    zeros16 = jnp.zeros((16,), jnp.float32)
    ebufs = (eb0, eb1)
    sems = (sem_in0, sem_in1)

    def zb_init(i, carry):
        zb[pl.ds(i * 16, 16)] = zeros16
        return carry

    lax.fori_loop(0, ACC_SLICE // 16, zb_init, 0)

    def in_copy(k, b):
        return pltpu.make_async_copy(
            edges_hbm.at[wid, pl.ds(k * CHUNK_ROWS * 3, CHUNK_ROWS * 3), :],
            ebufs[b], sems[b])

    def per_component(c, carry):
        pltpu.sync_copy(posed_hbm.at[c, 0, :], pc)
        pltpu.sync_copy(zb, acc.at[pl.ds(si * ACC_SLICE, ACC_SLICE)])
        plsc.subcore_barrier()

        in_copy(0, 0).start()
        in_copy(1, 1).start()

        def outer(i, carry2):
            for b in range(2):
                k = i * 2 + b
                eb = ebufs[b]
                in_copy(k, b).wait()
                for r in range(CHUNK_ROWS):
                    for g in range(ROW // 16):
                        sl = pl.ds(g * 16, 16)
                        idx = eb[r * 3 + 0, sl]
                        pv = plsc.load_gather(pc, [idx])
                        wv = plsc.bitcast(eb[r * 3 + 1, sl], jnp.float32)
                        valb[r, sl] = pv * wv
                descs = [
                    pltpu.async_copy(valb.at[r], acc.at[eb.at[r * 3 + 2, :]],
                                     sem_sc, add=True)
                    for r in range(CHUNK_ROWS)
                ]
                for d in descs:
                    d.wait()
                @pl.when(k + 2 < N_CHUNKS)
                def _():
                    in_copy(k + 2, b).start()
            return carry2

        lax.fori_loop(0, N_CHUNKS // 2, outer, 0)
        plsc.subcore_barrier()
        sl = pl.ds(si * ACC_SLICE, ACC_SLICE)
        pltpu.sync_copy(acc.at[sl], out_hbm.at[ci, c, 0, sl])
        plsc.subcore_barrier()
        return carry

    lax.fori_loop(0, 3, per_component, 0)


def _lap_call(posed_cm3, edges3):
    f = pl.kernel(
        _lap_body,
        out_type=jax.ShapeDtypeStruct((2, 3, 1, N_PAD), jnp.float32),
        mesh=_mesh(),
        compiler_params=pltpu.CompilerParams(needs_layout_passes=False),
        scratch_types=[
            pltpu.VMEM((N_PAD,), jnp.float32),
            pltpu.VMEM((CHUNK_ROWS * 3, ROW), jnp.int32),
            pltpu.VMEM((CHUNK_ROWS * 3, ROW), jnp.int32),
            pltpu.VMEM((CHUNK_ROWS, ROW), jnp.float32),
            pltpu.VMEM((ACC_SLICE,), jnp.float32),
            pltpu.VMEM_SHARED((N_PAD,), jnp.float32),
            pltpu.SemaphoreType.DMA,
            pltpu.SemaphoreType.DMA,
            pltpu.SemaphoreType.DMA,
        ],
    )
    return f(posed_cm3, edges3)


B_COMB = 2048


def _combine_body(p_ref, out_ref):
    out_ref[...] = p_ref[0] + p_ref[1]


def _combine_call(partials):
    grid = (N_PAD // B_COMB,)
    return pl.pallas_call(
        _combine_body,
        grid=grid,
        in_specs=[pl.BlockSpec((2, 3, B_COMB), lambda i: (0, 0, i))],
        out_specs=pl.BlockSpec((3, B_COMB), lambda i: (0, i)),
        out_shape=jax.ShapeDtypeStruct((3, N_PAD), jnp.float32),
    )(partials)


@jax.jit
def kernel(verts, joint_transforms, skin_w, lap_w, skin_idx, lap_src, lap_dst):
    n = verts.shape[0]
    pad = N_PAD - n
    v_t = jnp.pad(verts, ((0, pad), (0, 0))).T.reshape(3, 1, N_PAD)
    w_t = jnp.pad(skin_w, ((0, pad), (0, 0))).T.reshape(4, 1, N_PAD)
    i_t = jnp.pad(skin_idx.astype(jnp.int32),
                  ((0, pad), (0, 0))).T.reshape(4, 1, N_PAD)
    jt_flat = joint_transforms.reshape(-1)

    posed_cm3 = _skin_call(jt_flat, v_t, w_t, i_t)

    pad_e = E_PAD - N_LAP
    src_pad = (jnp.arange(pad_e, dtype=jnp.int32) * 16) % n
    dst_f = jnp.concatenate(
        [lap_dst.astype(jnp.int32), jnp.zeros((pad_e,), jnp.int32)]
    ).reshape(N_TILES, ROWS_PER_TILE, 1, ROW)
    src_f = jnp.concatenate(
        [lap_src.astype(jnp.int32), src_pad]
    ).reshape(N_TILES, ROWS_PER_TILE, 1, ROW)
    w_f = lax.bitcast_convert_type(
        jnp.concatenate([lap_w, jnp.zeros((pad_e,), jnp.float32)]), jnp.int32
    ).reshape(N_TILES, ROWS_PER_TILE, 1, ROW)
    edges3 = jnp.concatenate([dst_f, w_f, src_f], axis=2).reshape(
        N_TILES, ROWS_PER_TILE * 3, ROW)

    partials = _lap_call(posed_cm3, edges3)
    delta_cm = _combine_call(partials.reshape(2, 3, N_PAD))

    posed_cm = posed_cm3.reshape(3, N_PAD)
    posed = posed_cm[:, :n].T
    delta = delta_cm[:, :n].T
    return jnp.concatenate([posed, delta], axis=-1)


# speedup vs baseline: 1.0872x; 1.0872x over previous
import jax
import jax.numpy as jnp
from jax import lax
from jax.experimental import pallas as pl
from jax.experimental.pallas import tpu as pltpu
from jax.experimental.pallas import tpu_sc as plsc

N_VERTS = 100000
N_PAD = 102400
N_JOINTS = 64
K_INFL = 4
V_PER_TILE = N_PAD // 32
V_GROUPS = V_PER_TILE // 16

N_LAP = 1600000
ROW = 128
N_TILES = 32
ROWS_PER_TILE = 432
E_PAD = N_TILES * ROWS_PER_TILE * ROW
CHUNK_ROWS = 8
N_CHUNKS = ROWS_PER_TILE // CHUNK_ROWS
ACC_SLICE = N_PAD // 16


def _mesh():
    return plsc.VectorSubcoreMesh(core_axis_name="c", subcore_axis_name="s",
                                  num_cores=2, num_subcores=16)


def _skin_body(jt_hbm, v_hbm, w_hbm, i_hbm, out_hbm,
               jtbl, vs0, vs1, vs2, ws0, ws1, ws2, ws3,
               is0, is1, is2, is3, ps0, ps1, ps2, sem):
    ci = lax.axis_index("c")
    si = lax.axis_index("s")
    wid = ci * 16 + si
    v0 = wid * V_PER_TILE

    descs = [pltpu.async_copy(jt_hbm, jtbl, sem)]
    for c, ref in enumerate((vs0, vs1, vs2)):
        descs.append(
            pltpu.async_copy(v_hbm.at[c, 0, pl.ds(v0, V_PER_TILE)], ref, sem))
    for k, ref in enumerate((ws0, ws1, ws2, ws3)):
        descs.append(
            pltpu.async_copy(w_hbm.at[k, 0, pl.ds(v0, V_PER_TILE)], ref, sem))
    for k, ref in enumerate((is0, is1, is2, is3)):
        descs.append(
            pltpu.async_copy(i_hbm.at[k, 0, pl.ds(v0, V_PER_TILE)], ref, sem))
    for d in descs:
        d.wait()
    wrefs = (ws0, ws1, ws2, ws3)
    irefs = (is0, is1, is2, is3)

    def group(g, carry):
        sl = pl.ds(g * 16, 16)
        x = vs0[sl]
        y = vs1[sl]
        z = vs2[sl]
        w = [wrefs[k][sl] for k in range(K_INFL)]
        s = ((w[0] + w[1]) + (w[2] + w[3])) + 1e-8
        rs = 1.0 / s
        acc0 = jnp.zeros((16,), jnp.float32)
        acc1 = jnp.zeros((16,), jnp.float32)
        acc2 = jnp.zeros((16,), jnp.float32)
        for k in range(K_INFL):
            jk = irefs[k][sl]
            jb = jk * 16
            wk = w[k] * rs
            e = [plsc.load_gather(jtbl, [jb + t]) for t in range(12)]
            acc0 = acc0 + wk * (e[0] * x + e[1] * y + e[2] * z + e[3])
            acc1 = acc1 + wk * (e[4] * x + e[5] * y + e[6] * z + e[7])
            acc2 = acc2 + wk * (e[8] * x + e[9] * y + e[10] * z + e[11])
        ps0[sl] = acc0
        ps1[sl] = acc1
        ps2[sl] = acc2
        return carry

    lax.fori_loop(0, V_GROUPS, group, 0)
    for c, ps in enumerate((ps0, ps1, ps2)):
        pltpu.sync_copy(ps, out_hbm.at[c, 0, pl.ds(v0, V_PER_TILE)])


def _skin_call(jt_flat, v_flat, w_flat, i_flat):
    f = pl.kernel(
        _skin_body,
        out_type=jax.ShapeDtypeStruct((3, 1, N_PAD), jnp.float32),
        mesh=_mesh(),
        compiler_params=pltpu.CompilerParams(needs_layout_passes=False),
        scratch_types=(
            [pltpu.VMEM((N_JOINTS * 16,), jnp.float32)]
            + [pltpu.VMEM((V_PER_TILE,), jnp.float32) for _ in range(3)]
            + [pltpu.VMEM((V_PER_TILE,), jnp.float32) for _ in range(4)]
            + [pltpu.VMEM((V_PER_TILE,), jnp.int32) for _ in range(4)]
            + [pltpu.VMEM((V_PER_TILE,), jnp.float32) for _ in range(3)]
            + [pltpu.SemaphoreType.DMA]
        ),
    )
    return f(jt_flat, v_flat, w_flat, i_flat)


def _lap_body(posed_hbm, edges_hbm, out_hbm,
              pc, eb0, eb1, eb2, valb0, valb1, zb, acc,
              sem_in0, sem_in1, sem_in2, sem_sc0, sem_sc1):
    ci = lax.axis_index("c")
    si = lax.axis_index("s")
    wid = ci * 16 + si

    zeros16 = jnp.zeros((16,), jnp.float32)
    ebufs = (eb0, eb1, eb2)
    isems = (sem_in0, sem_in1, sem_in2)
    vbufs = (valb0, valb1)
    ssems = (sem_sc0, sem_sc1)

    def zb_init(i, carry):
        zb[pl.ds(i * 16, 16)] = zeros16
        return carry

    lax.fori_loop(0, ACC_SLICE // 32, zb_init, 0)

    def in_copy(k, j):
        return pltpu.make_async_copy(
            edges_hbm.at[wid, pl.ds(k * CHUNK_ROWS * 3, CHUNK_ROWS * 3), :],
            ebufs[j], isems[j])

    def scat(r, j, p):
        return pltpu.make_async_copy(
            vbufs[p].at[r], acc.at[ebufs[j].at[r * 3 + 2, :]], ssems[p])

    def per_component(c, carry):
        pltpu.sync_copy(posed_hbm.at[c, 0, :], pc)
        half = ACC_SLICE // 2
        pltpu.sync_copy(zb, acc.at[pl.ds(si * ACC_SLICE, half)])
        pltpu.sync_copy(zb, acc.at[pl.ds(si * ACC_SLICE + half, half)])
        plsc.subcore_barrier()

        in_copy(0, 0).start()
        in_copy(1, 1).start()

        def outer(i, carry2):
            k6 = i * 6
            for b in range(6):
                k = k6 + b
                j = b % 3
                p = b % 2
                eb = ebufs[j]
                vb = vbufs[p]
                in_copy(k, j).wait()
                for r in range(CHUNK_ROWS):
                    for g in range(ROW // 16):
                        sl = pl.ds(g * 16, 16)
                        idx = eb[r * 3 + 0, sl]
                        pv = plsc.load_gather(pc, [idx])
                        wv = plsc.bitcast(eb[r * 3 + 1, sl], jnp.float32)
                        vb[r, sl] = pv * wv
                @pl.when(k >= 1)
                def _():
                    for r in range(CHUNK_ROWS):
                        scat(r, (j + 2) % 3, 1 - p).wait()
                @pl.when(k + 2 < N_CHUNKS)
                def _():
                    in_copy(k + 2, (j + 2) % 3).start()
                for r in range(CHUNK_ROWS):
                    d = scat(r, j, p)
                    d.start(add=True)
            return carry2

        lax.fori_loop(0, N_CHUNKS // 6, outer, 0)
        for r in range(CHUNK_ROWS):
            scat(r, 2, 1).wait()
        plsc.subcore_barrier()
        sl = pl.ds(si * ACC_SLICE, ACC_SLICE)
        pltpu.sync_copy(acc.at[sl], out_hbm.at[ci, c, 0, sl])
        plsc.subcore_barrier()
        return carry

    lax.fori_loop(0, 3, per_component, 0)


def _lap_call(posed_cm3, edges3):
    f = pl.kernel(
        _lap_body,
        out_type=jax.ShapeDtypeStruct((2, 3, 1, N_PAD), jnp.float32),
        mesh=_mesh(),
        compiler_params=pltpu.CompilerParams(needs_layout_passes=False),
        scratch_types=[
            pltpu.VMEM((N_PAD,), jnp.float32),
            pltpu.VMEM((CHUNK_ROWS * 3, ROW), jnp.int32),
            pltpu.VMEM((CHUNK_ROWS * 3, ROW), jnp.int32),
            pltpu.VMEM((CHUNK_ROWS * 3, ROW), jnp.int32),
            pltpu.VMEM((CHUNK_ROWS, ROW), jnp.float32),
            pltpu.VMEM((CHUNK_ROWS, ROW), jnp.float32),
            pltpu.VMEM((ACC_SLICE // 2,), jnp.float32),
            pltpu.VMEM_SHARED((N_PAD,), jnp.float32),
            pltpu.SemaphoreType.DMA,
            pltpu.SemaphoreType.DMA,
            pltpu.SemaphoreType.DMA,
            pltpu.SemaphoreType.DMA,
            pltpu.SemaphoreType.DMA,
        ],
    )
    return f(posed_cm3, edges3)


B_COMB = 2048


def _combine_body(p_ref, out_ref):
    out_ref[...] = p_ref[0] + p_ref[1]


def _combine_call(partials):
    grid = (N_PAD // B_COMB,)
    return pl.pallas_call(
        _combine_body,
        grid=grid,
        in_specs=[pl.BlockSpec((2, 3, B_COMB), lambda i: (0, 0, i))],
        out_specs=pl.BlockSpec((3, B_COMB), lambda i: (0, i)),
        out_shape=jax.ShapeDtypeStruct((3, N_PAD), jnp.float32),
    )(partials)


@jax.jit
def kernel(verts, joint_transforms, skin_w, lap_w, skin_idx, lap_src, lap_dst):
    n = verts.shape[0]
    pad = N_PAD - n
    v_t = jnp.pad(verts, ((0, pad), (0, 0))).T.reshape(3, 1, N_PAD)
    w_t = jnp.pad(skin_w, ((0, pad), (0, 0))).T.reshape(4, 1, N_PAD)
    i_t = jnp.pad(skin_idx.astype(jnp.int32),
                  ((0, pad), (0, 0))).T.reshape(4, 1, N_PAD)
    jt_flat = joint_transforms.reshape(-1)

    posed_cm3 = _skin_call(jt_flat, v_t, w_t, i_t)

    pad_e = E_PAD - N_LAP
    src_pad = (jnp.arange(pad_e, dtype=jnp.int32) * 16) % n
    dst_f = jnp.concatenate(
        [lap_dst.astype(jnp.int32), jnp.zeros((pad_e,), jnp.int32)]
    ).reshape(N_TILES, ROWS_PER_TILE, 1, ROW)
    src_f = jnp.concatenate(
        [lap_src.astype(jnp.int32), src_pad]
    ).reshape(N_TILES, ROWS_PER_TILE, 1, ROW)
    w_f = lax.bitcast_convert_type(
        jnp.concatenate([lap_w, jnp.zeros((pad_e,), jnp.float32)]), jnp.int32
    ).reshape(N_TILES, ROWS_PER_TILE, 1, ROW)
    edges3 = jnp.concatenate([dst_f, w_f, src_f], axis=2).reshape(
        N_TILES, ROWS_PER_TILE * 3, ROW)

    partials = _lap_call(posed_cm3, edges3)
    delta_cm = _combine_call(partials.reshape(2, 3, N_PAD))

    posed_cm = posed_cm3.reshape(3, N_PAD)
    posed = posed_cm[:, :n].T
    delta = delta_cm[:, :n].T
    return jnp.concatenate([posed, delta], axis=-1)
